# baseline (speedup 1.0000x reference)
import jax
import jax.numpy as jnp
import numpy as np
from jax.experimental import pallas as pl
from jax.experimental.pallas import tpu as pltpu

B, T, N = 8, 8, 1024
CODE_SIZE, GRAPH_SIZE, HIDDEN, ATT = 48, 32, 128, 32
CATE_NUM, TEXT_DIM, OUT_SIZE = 20, 300, 1
CAT_SIZE = HIDDEN + HIDDEN // 4 + HIDDEN // 4 + HIDDEN // 8 + TEXT_DIM


def _dot(a, b):
    return jax.lax.dot_general(a, b, (((1,), (0,)), ((), ())),
                               preferred_element_type=jnp.float32)


def _dot_t(a, b):
    return jax.lax.dot_general(a, b, (((1,), (1,)), ((), ())),
                               preferred_element_type=jnp.float32)


def _dot_lt(a, b):
    return jax.lax.dot_general(a, b, (((0,), (0,)), ((), ())),
                               preferred_element_type=jnp.float32)


KC = 256


def _masked_max(h, m):
    v = jnp.max(m * h + (1.0 - m) * (-1e9), axis=0, keepdims=True)
    return jnp.max(m) * v


def _fwd_kernel(cx_ref, nb_ref, dv_ref, cate_ref, tf_ref, tm_ref,
                ug_ref, ua_ref, uc_ref, cdoh_ref,
                c_emb_ref, n_emb_ref, u_emb_ref, cate_emb_ref,
                adj_ref, cate_adj_ref, w_g_ref, b_g_ref,
                w_ir_ref, w_iz_ref, w_in_ref,
                w_hr_ref, w_hz_ref, w_hn_ref,
                b_r_ref, b_z_ref, b_n_ref,
                w_q_ref, w_k_ref, w_v_ref, w_sa_ref,
                w_da_ref, b_da_ref, ctx_da_ref,
                w_da2_ref, b_da2_ref, ctx_da2_ref,
                us_g_ref, us_a_ref, us_c_ref,
                o_ref):
    cx = cx_ref[0]
    nb = nb_ref[0]
    dv = dv_ref[0]
    cate = cate_ref[0]
    tf = tf_ref[0]
    tm = tm_ref[0]

    c_emb = c_emb_ref[...]
    n_emb = n_emb_ref[...]
    u_emb = u_emb_ref[...]
    cate_emb = cate_emb_ref[...]
    adj = adj_ref[...]
    cate_adj = cate_adj_ref[...]
    w_g = w_g_ref[...]
    b_g = b_g_ref[...]
    cdoh = cdoh_ref[...]

    tri = (jax.lax.broadcasted_iota(jnp.int32, (N, N), 0)
           >= jax.lax.broadcasted_iota(jnp.int32, (N, N), 1)).astype(jnp.float32)
    rowi = jax.lax.broadcasted_iota(jnp.int32, (KC, N), 0)

    def step(t, h, no_prev):
        c = cx[t][:, None]
        n = nb[t][:, None]
        ce = c * c_emb
        ne = n * n_emb
        s = _dot(adj, ce + ne)
        co = jax.nn.leaky_relu(_dot(ce + c * s, w_g) + b_g)
        no = jax.nn.leaky_relu(_dot(ne + n * s, w_g) + b_g)

        ct = cate[t][:, None]
        ca = ct * cate_emb
        cao = jax.nn.leaky_relu(_dot(ca + ct * _dot(cate_adj, ca), w_g) + b_g)

        m1 = dv[t, 0][:, None]
        m2 = dv[t, 1][:, None]
        m3 = dv[t, 2][:, None]

        r = jax.nn.sigmoid(_dot(co, w_ir_ref[...]) + _dot(h, w_hr_ref[...]) + b_r_ref[...])
        z = jax.nn.sigmoid(_dot(co, w_iz_ref[...]) + _dot(h, w_hz_ref[...]) + b_z_ref[...])
        g = jnp.tanh(_dot(co, w_in_ref[...]) + r * _dot(h, w_hn_ref[...]) + b_n_ref[...])
        h_m1 = (1.0 - z) * g + z * h

        out = _masked_max(h_m1, m1)
        h_new = m1 * h_m1
        if t > 0:
            m23 = jnp.maximum(m2, m3)
            cao_pc = _dot(cdoh, cao)
            q = m2 * no_prev + (1.0 - m2) * m3 * (u_emb + cao_pc)
            pos = _dot(tri, m23)
            posi = (pos - 1.0).astype(jnp.int32)
            G = (rowi == posi.T).astype(jnp.float32) * m23.T
            cnt = jnp.max(posi) + 1
            qc = _dot(G, q)
            qhc = _dot(qc, w_q_ref[...]) * (1.0 / np.sqrt(ATT))
            kh = _dot(q, w_k_ref[...])
            vh = _dot(co, w_v_ref[...])
            e = jnp.exp(_dot_t(qhc, kh)) * m23.T
            srow = jnp.sum(e, axis=-1, keepdims=True)
            hc = jnp.tanh(_dot(e, vh) / jnp.maximum(srow, 1e-30))
            h_new = _dot_lt(G, hc) + (1.0 - m23) * h_new
            maskc = (rowi[:, :1] < cnt).astype(jnp.float32)

            out = out + _masked_max(hc, maskc)
        return h_new, no, out

    h, no_prev, out0 = step(0, jnp.zeros((N, HIDDEN), jnp.float32),
                            jnp.zeros((N, GRAPH_SIZE), jnp.float32))
    outs = [out0]
    for t in range(1, T):
        h_new, no, out = step(t, h, no_prev)
        v = tm[0, t]
        h = v * h_new + (1.0 - v) * h
        no_prev = v * no + (1.0 - v) * no_prev
        outs.append(v * out)

    H = jnp.concatenate(outs, axis=0)
    sa = _dot_t(_dot(H, w_sa_ref[...]), H) * (1.0 / np.sqrt(HIDDEN))
    sa = jnp.where(tm > 0, sa, -1e9)
    Hs = _dot(jax.nn.softmax(sa, axis=-1), H)

    u = jnp.tanh(_dot(Hs, w_da_ref[...]) + b_da_ref[...])
    sc = _dot(u, ctx_da_ref[...])
    sc = jnp.where(tm.T > 0, sc, -1e9)
    a = jax.nn.softmax(sc, axis=0)
    seq_pool = _dot_t(a.T, Hs.T)

    u2 = jnp.tanh(_dot(tf, w_da2_ref[...]) + b_da2_ref[...])
    sc2 = _dot(u2, ctx_da2_ref[...])
    sc2 = jnp.where(tm.T > 0, sc2, -1e9)
    a2 = jax.nn.softmax(sc2, axis=0)
    txt_pool = _dot_t(a2.T, tf.T)

    ue_g = _dot(ug_ref[0], us_g_ref[...])
    ue_a = _dot(ua_ref[0], us_a_ref[...])
    ue_c = _dot(uc_ref[0], us_c_ref[...])

    o_ref[...] = jnp.concatenate(
        [ue_g, ue_a, ue_c, txt_pool, seq_pool], axis=1).reshape(1, 1, CAT_SIZE)


def _bn_kernel(o_ref, gamma_ref, beta_ref, wc_ref, bc_ref, out_ref):
    O = o_ref[...]
    mean = jnp.mean(O, axis=0, keepdims=True)
    var = jnp.mean((O - mean) * (O - mean), axis=0, keepdims=True)
    On = (O - mean) / jnp.sqrt(var + 1e-5) * gamma_ref[...] + beta_ref[...]
    out_ref[...] = _dot(On, wc_ref[...]) + bc_ref[...]


def _bcast_spec(shape):
    nd = len(shape)
    return pl.BlockSpec(shape, lambda i, _nd=nd: (0,) * _nd)


@jax.jit
def kernel(code_x, divided, neighbors, lens, user, cate, text_features,
           event_types, cate_dict, params):
    p = params
    f32 = jnp.float32

    dv = jnp.transpose(divided, (0, 1, 3, 2))
    tm = (jnp.arange(T)[None, :] < lens[:, None]).astype(f32).reshape(B, 1, T)
    ug = (user[:, 0:1] == jnp.arange(2)[None, :]).astype(f32).reshape(B, 1, 2)
    ua = (user[:, 1:2] == jnp.arange(13)[None, :]).astype(f32).reshape(B, 1, 13)
    uc = (user[:, 2:3] == jnp.arange(10)[None, :]).astype(f32).reshape(B, 1, 10)
    cdoh = (cate_dict[:, None] == jnp.arange(CATE_NUM)[None, :]).astype(f32)

    row = lambda x: x.reshape(1, -1)
    col = lambda x: x.reshape(-1, 1)

    per_batch = [
        (code_x, (1, T, N)),
        (neighbors, (1, T, N)),
        (dv, (1, T, 3, N)),
        (cate, (1, T, CATE_NUM)),
        (text_features, (1, T, TEXT_DIM)),
        (tm, (1, 1, T)),
        (ug, (1, 1, 2)),
        (ua, (1, 1, 13)),
        (uc, (1, 1, 10)),
    ]
    bcast = [
        cdoh, p['c_emb'], p['n_emb'], p['u_emb'], p['cate_emb'],
        p['adj'], p['cate_adj'], p['W_g'], row(p['b_g']),
        p['W_ir'], p['W_iz'], p['W_in'],
        p['W_hr'], p['W_hz'], p['W_hn'],
        row(p['b_r']), row(p['b_z']), row(p['b_n']),
        p['W_q'], p['W_k'], p['W_v'], p['W_sa'],
        p['W_da'], row(p['b_da']), col(p['ctx_da']),
        p['W_da2'], row(p['b_da2']), col(p['ctx_da2']),
        p['us_gender'], p['us_age'], p['us_cluster'],
    ]

    in_specs = [pl.BlockSpec(bs, lambda i, _nd=len(bs): (i,) + (0,) * (_nd - 1))
                for _, bs in per_batch]
    in_specs += [_bcast_spec(a.shape) for a in bcast]

    O = pl.pallas_call(
        _fwd_kernel,
        grid=(B,),
        in_specs=in_specs,
        out_specs=pl.BlockSpec((1, 1, CAT_SIZE), lambda i: (i, 0, 0)),
        out_shape=jax.ShapeDtypeStruct((B, 1, CAT_SIZE), f32),
        compiler_params=pltpu.CompilerParams(
            dimension_semantics=("arbitrary",)),
    )(*[a for a, _ in per_batch], *bcast)

    out = pl.pallas_call(
        _bn_kernel,
        out_shape=jax.ShapeDtypeStruct((B, OUT_SIZE), f32),
    )(O.reshape(B, CAT_SIZE), row(p['bn_gamma']), row(p['bn_beta']),
      p['W_c'], row(p['b_c']))
    return out


# speedup vs baseline: 1.2024x; 1.2024x over previous
import jax
import jax.numpy as jnp
import numpy as np
from jax.experimental import pallas as pl
from jax.experimental.pallas import tpu as pltpu

B, T, N = 8, 8, 1024
CODE_SIZE, GRAPH_SIZE, HIDDEN, ATT = 48, 32, 128, 32
CATE_NUM, TEXT_DIM, OUT_SIZE = 20, 300, 1
CAT_SIZE = HIDDEN + HIDDEN // 4 + HIDDEN // 4 + HIDDEN // 8 + TEXT_DIM


def _dot(a, b):
    return jax.lax.dot_general(a, b, (((1,), (0,)), ((), ())),
                               preferred_element_type=jnp.float32)


def _dot_t(a, b):
    return jax.lax.dot_general(a, b, (((1,), (1,)), ((), ())),
                               preferred_element_type=jnp.float32)


def _bf(x):
    return x.astype(jnp.bfloat16)


def _masked_max(h, m):
    v = jnp.max(m * h + (1.0 - m) * (-1e9), axis=0, keepdims=True)
    return jnp.max(m) * v


def _fwd_kernel(cx_ref, nb_ref, dv_ref, cate_ref, tf_ref, tm_ref,
                ug_ref, ua_ref, uc_ref, cdoh_ref,
                c_emb_ref, n_emb_ref, u_emb_ref, cate_emb_ref,
                adj_ref, cate_adj_ref, w_g_ref, b_g_ref,
                w_ir_ref, w_iz_ref, w_in_ref,
                w_hr_ref, w_hz_ref, w_hn_ref,
                b_r_ref, b_z_ref, b_n_ref,
                w_q_ref, w_k_ref, w_v_ref, w_sa_ref,
                w_da_ref, b_da_ref, ctx_da_ref,
                w_da2_ref, b_da2_ref, ctx_da2_ref,
                us_g_ref, us_a_ref, us_c_ref,
                o_ref):
    cx = cx_ref[0]
    nb = nb_ref[0]
    dv = dv_ref[0]
    cate = cate_ref[0]
    tf = tf_ref[0]
    tm = tm_ref[0]

    c_emb = c_emb_ref[...]
    n_emb = n_emb_ref[...]
    u_emb = u_emb_ref[...]
    cate_emb = cate_emb_ref[...]
    adj = adj_ref[...]
    cate_adj = cate_adj_ref[...]
    w_g = w_g_ref[...]
    b_g = b_g_ref[...]
    cdoh = cdoh_ref[...]

    def step(t, h, no_prev):
        c = cx[t][:, None]
        n = nb[t][:, None]
        ce = c * c_emb
        ne = n * n_emb
        s = _dot(adj, ce + ne)
        co = jax.nn.leaky_relu(_dot(ce + c * s, w_g) + b_g)
        no = jax.nn.leaky_relu(_dot(ne + n * s, w_g) + b_g)

        ct = cate[t][:, None]
        ca = ct * cate_emb
        cao = jax.nn.leaky_relu(_dot(ca + ct * _dot(cate_adj, ca), w_g) + b_g)

        m1 = dv[t, 0][:, None]
        m2 = dv[t, 1][:, None]
        m3 = dv[t, 2][:, None]

        r = jax.nn.sigmoid(_dot(co, w_ir_ref[...]) + _dot(h, w_hr_ref[...]) + b_r_ref[...])
        z = jax.nn.sigmoid(_dot(co, w_iz_ref[...]) + _dot(h, w_hz_ref[...]) + b_z_ref[...])
        g = jnp.tanh(_dot(co, w_in_ref[...]) + r * _dot(h, w_hn_ref[...]) + b_n_ref[...])
        h_m1 = (1.0 - z) * g + z * h

        out = _masked_max(h_m1, m1)
        h_new = m1 * h_m1
        if t > 0:
            m23 = jnp.maximum(m2, m3)
            cao_pc = _dot(cdoh, cao)
            q = m2 * no_prev + (1.0 - m2) * m3 * (u_emb + cao_pc)
            qh = _dot(q, w_q_ref[...]) * (1.0 / np.sqrt(ATT))
            kh = _dot(q, w_k_ref[...])
            vh = _dot(co, w_v_ref[...])
            e = jnp.exp(_dot_t(_bf(qh), _bf(kh))) * m23.T
            srow = jnp.sum(e, axis=-1, keepdims=True)
            h_m23 = jnp.tanh(_dot(_bf(e), _bf(vh)) / jnp.maximum(srow, 1e-30))
            h_new = m23 * h_m23 + (1.0 - m23) * h_new
            out = out + _masked_max(h_m23, m23)
        return h_new, no, out

    h, no_prev, out0 = step(0, jnp.zeros((N, HIDDEN), jnp.float32),
                            jnp.zeros((N, GRAPH_SIZE), jnp.float32))
    outs = [out0]
    for t in range(1, T):
        h_new, no, out = step(t, h, no_prev)
        v = tm[0, t]
        h = v * h_new + (1.0 - v) * h
        no_prev = v * no + (1.0 - v) * no_prev
        outs.append(v * out)

    H = jnp.concatenate(outs, axis=0)
    sa = _dot_t(_dot(H, w_sa_ref[...]), H) * (1.0 / np.sqrt(HIDDEN))
    sa = jnp.where(tm > 0, sa, -1e9)
    Hs = _dot(jax.nn.softmax(sa, axis=-1), H)

    u = jnp.tanh(_dot(Hs, w_da_ref[...]) + b_da_ref[...])
    sc = _dot(u, ctx_da_ref[...])
    sc = jnp.where(tm.T > 0, sc, -1e9)
    a = jax.nn.softmax(sc, axis=0)
    seq_pool = _dot_t(a.T, Hs.T)

    u2 = jnp.tanh(_dot(tf, w_da2_ref[...]) + b_da2_ref[...])
    sc2 = _dot(u2, ctx_da2_ref[...])
    sc2 = jnp.where(tm.T > 0, sc2, -1e9)
    a2 = jax.nn.softmax(sc2, axis=0)
    txt_pool = _dot_t(a2.T, tf.T)

    ue_g = _dot(ug_ref[0], us_g_ref[...])
    ue_a = _dot(ua_ref[0], us_a_ref[...])
    ue_c = _dot(uc_ref[0], us_c_ref[...])

    o_ref[...] = jnp.concatenate(
        [ue_g, ue_a, ue_c, txt_pool, seq_pool], axis=1).reshape(1, 1, CAT_SIZE)


def _bn_kernel(o_ref, gamma_ref, beta_ref, wc_ref, bc_ref, out_ref):
    O = o_ref[...]
    mean = jnp.mean(O, axis=0, keepdims=True)
    var = jnp.mean((O - mean) * (O - mean), axis=0, keepdims=True)
    On = (O - mean) / jnp.sqrt(var + 1e-5) * gamma_ref[...] + beta_ref[...]
    out_ref[...] = _dot(On, wc_ref[...]) + bc_ref[...]


def _bcast_spec(shape):
    nd = len(shape)
    return pl.BlockSpec(shape, lambda i, _nd=nd: (0,) * _nd)


@jax.jit
def kernel(code_x, divided, neighbors, lens, user, cate, text_features,
           event_types, cate_dict, params):
    p = params
    f32 = jnp.float32

    dv = jnp.transpose(divided, (0, 1, 3, 2))
    tm = (jnp.arange(T)[None, :] < lens[:, None]).astype(f32).reshape(B, 1, T)
    ug = (user[:, 0:1] == jnp.arange(2)[None, :]).astype(f32).reshape(B, 1, 2)
    ua = (user[:, 1:2] == jnp.arange(13)[None, :]).astype(f32).reshape(B, 1, 13)
    uc = (user[:, 2:3] == jnp.arange(10)[None, :]).astype(f32).reshape(B, 1, 10)
    cdoh = (cate_dict[:, None] == jnp.arange(CATE_NUM)[None, :]).astype(f32)

    row = lambda x: x.reshape(1, -1)
    col = lambda x: x.reshape(-1, 1)

    per_batch = [
        (code_x, (1, T, N)),
        (neighbors, (1, T, N)),
        (dv, (1, T, 3, N)),
        (cate, (1, T, CATE_NUM)),
        (text_features, (1, T, TEXT_DIM)),
        (tm, (1, 1, T)),
        (ug, (1, 1, 2)),
        (ua, (1, 1, 13)),
        (uc, (1, 1, 10)),
    ]
    bcast = [
        cdoh, p['c_emb'], p['n_emb'], p['u_emb'], p['cate_emb'],
        p['adj'], p['cate_adj'], p['W_g'], row(p['b_g']),
        p['W_ir'], p['W_iz'], p['W_in'],
        p['W_hr'], p['W_hz'], p['W_hn'],
        row(p['b_r']), row(p['b_z']), row(p['b_n']),
        p['W_q'], p['W_k'], p['W_v'], p['W_sa'],
        p['W_da'], row(p['b_da']), col(p['ctx_da']),
        p['W_da2'], row(p['b_da2']), col(p['ctx_da2']),
        p['us_gender'], p['us_age'], p['us_cluster'],
    ]

    in_specs = [pl.BlockSpec(bs, lambda i, _nd=len(bs): (i,) + (0,) * (_nd - 1))
                for _, bs in per_batch]
    in_specs += [_bcast_spec(a.shape) for a in bcast]

    O = pl.pallas_call(
        _fwd_kernel,
        grid=(B,),
        in_specs=in_specs,
        out_specs=pl.BlockSpec((1, 1, CAT_SIZE), lambda i: (i, 0, 0)),
        out_shape=jax.ShapeDtypeStruct((B, 1, CAT_SIZE), f32),
        compiler_params=pltpu.CompilerParams(
            dimension_semantics=("arbitrary",)),
    )(*[a for a, _ in per_batch], *bcast)

    out = pl.pallas_call(
        _bn_kernel,
        out_shape=jax.ShapeDtypeStruct((B, OUT_SIZE), f32),
    )(O.reshape(B, CAT_SIZE), row(p['bn_gamma']), row(p['bn_beta']),
      p['W_c'], row(p['b_c']))
    return out


# speedup vs baseline: 1.2224x; 1.0167x over previous
import jax
import jax.numpy as jnp
import numpy as np
from jax.experimental import pallas as pl
from jax.experimental.pallas import tpu as pltpu

B, T, N = 8, 8, 1024
CODE_SIZE, GRAPH_SIZE, HIDDEN, ATT = 48, 32, 128, 32
CATE_NUM, TEXT_DIM, OUT_SIZE = 20, 300, 1
CAT_SIZE = HIDDEN + HIDDEN // 4 + HIDDEN // 4 + HIDDEN // 8 + TEXT_DIM


def _dot(a, b):
    return jax.lax.dot_general(a, b, (((1,), (0,)), ((), ())),
                               preferred_element_type=jnp.float32)


def _dot_t(a, b):
    return jax.lax.dot_general(a, b, (((1,), (1,)), ((), ())),
                               preferred_element_type=jnp.float32)


def _masked_max(h, m):
    v = jnp.max(m * h + (1.0 - m) * (-1e9), axis=0, keepdims=True)
    return jnp.max(m) * v


def _fwd_kernel(cx_ref, nb_ref, dv_ref, cate_ref, tf_ref, tm_ref,
                ug_ref, ua_ref, uc_ref, cdoh_ref,
                c_emb_ref, n_emb_ref, u_emb_ref, cate_emb_ref,
                adj_ref, cate_adj_ref, w_g_ref, b_g_ref,
                w_ir_ref, w_iz_ref, w_in_ref,
                w_hr_ref, w_hz_ref, w_hn_ref,
                b_r_ref, b_z_ref, b_n_ref,
                w_q_ref, w_k_ref, w_v_ref, w_sa_ref,
                w_da_ref, b_da_ref, ctx_da_ref,
                w_da2_ref, b_da2_ref, ctx_da2_ref,
                us_g_ref, us_a_ref, us_c_ref,
                o_ref):
    cx = cx_ref[0]
    nb = nb_ref[0]
    dv = dv_ref[0]
    cate = cate_ref[0]
    tf = tf_ref[0]
    tm = tm_ref[0]

    c_emb = c_emb_ref[...]
    n_emb = n_emb_ref[...]
    u_emb = u_emb_ref[...]
    cate_emb = cate_emb_ref[...]
    adj = adj_ref[...]
    cate_adj = cate_adj_ref[...]
    w_g = w_g_ref[...]
    b_g = b_g_ref[...]
    cdoh = cdoh_ref[...]

    def step(t, h, no_prev):
        c = cx[t][:, None]
        n = nb[t][:, None]
        ce = c * c_emb
        ne = n * n_emb
        s = _dot(adj, ce + ne)
        co = jax.nn.leaky_relu(_dot(ce + c * s, w_g) + b_g)
        no = jax.nn.leaky_relu(_dot(ne + n * s, w_g) + b_g)

        ct = cate[t][:, None]
        ca = ct * cate_emb
        cao = jax.nn.leaky_relu(_dot(ca + ct * _dot(cate_adj, ca), w_g) + b_g)

        m1 = dv[t, 0][:, None]
        m2 = dv[t, 1][:, None]
        m3 = dv[t, 2][:, None]

        r = jax.nn.sigmoid(_dot(co, w_ir_ref[...]) + _dot(h, w_hr_ref[...]) + b_r_ref[...])
        z = jax.nn.sigmoid(_dot(co, w_iz_ref[...]) + _dot(h, w_hz_ref[...]) + b_z_ref[...])
        g = jnp.tanh(_dot(co, w_in_ref[...]) + r * _dot(h, w_hn_ref[...]) + b_n_ref[...])
        h_m1 = (1.0 - z) * g + z * h

        out = _masked_max(h_m1, m1)
        h_new = m1 * h_m1
        if t > 0:
            m23 = jnp.maximum(m2, m3)
            cao_pc = _dot(cdoh, cao)
            q = m2 * no_prev + (1.0 - m2) * m3 * (u_emb + cao_pc)
            qh = _dot(q, w_q_ref[...]) * (1.0 / np.sqrt(ATT))
            kh = _dot(q, w_k_ref[...])
            vh = _dot(co, w_v_ref[...])
            e = jnp.exp(_dot_t(qh, kh)) * m23.T
            vh1 = jnp.concatenate([vh, jnp.ones((N, 1), jnp.float32)], axis=1)
            hs = _dot(e, vh1)
            h_m23 = jnp.tanh(hs[:, :HIDDEN] /
                             jnp.maximum(hs[:, HIDDEN:HIDDEN + 1], 1e-30))
            h_new = m23 * h_m23 + (1.0 - m23) * h_new
            out = out + _masked_max(h_m23, m23)
        return h_new, no, out

    h, no_prev, out0 = step(0, jnp.zeros((N, HIDDEN), jnp.float32),
                            jnp.zeros((N, GRAPH_SIZE), jnp.float32))
    outs = [out0]
    for t in range(1, T):
        h_new, no, out = step(t, h, no_prev)
        v = tm[0, t]
        h = v * h_new + (1.0 - v) * h
        no_prev = v * no + (1.0 - v) * no_prev
        outs.append(v * out)

    H = jnp.concatenate(outs, axis=0)
    sa = _dot_t(_dot(H, w_sa_ref[...]), H) * (1.0 / np.sqrt(HIDDEN))
    sa = jnp.where(tm > 0, sa, -1e9)
    Hs = _dot(jax.nn.softmax(sa, axis=-1), H)

    u = jnp.tanh(_dot(Hs, w_da_ref[...]) + b_da_ref[...])
    sc = _dot(u, ctx_da_ref[...])
    sc = jnp.where(tm.T > 0, sc, -1e9)
    a = jax.nn.softmax(sc, axis=0)
    seq_pool = _dot_t(a.T, Hs.T)

    u2 = jnp.tanh(_dot(tf, w_da2_ref[...]) + b_da2_ref[...])
    sc2 = _dot(u2, ctx_da2_ref[...])
    sc2 = jnp.where(tm.T > 0, sc2, -1e9)
    a2 = jax.nn.softmax(sc2, axis=0)
    txt_pool = _dot_t(a2.T, tf.T)

    ue_g = _dot(ug_ref[0], us_g_ref[...])
    ue_a = _dot(ua_ref[0], us_a_ref[...])
    ue_c = _dot(uc_ref[0], us_c_ref[...])

    o_ref[...] = jnp.concatenate(
        [ue_g, ue_a, ue_c, txt_pool, seq_pool], axis=1).reshape(1, 1, CAT_SIZE)


def _bn_kernel(o_ref, gamma_ref, beta_ref, wc_ref, bc_ref, out_ref):
    O = o_ref[...]
    mean = jnp.mean(O, axis=0, keepdims=True)
    var = jnp.mean((O - mean) * (O - mean), axis=0, keepdims=True)
    On = (O - mean) / jnp.sqrt(var + 1e-5) * gamma_ref[...] + beta_ref[...]
    out_ref[...] = _dot(On, wc_ref[...]) + bc_ref[...]


def _bcast_spec(shape):
    nd = len(shape)
    return pl.BlockSpec(shape, lambda i, _nd=nd: (0,) * _nd)


@jax.jit
def kernel(code_x, divided, neighbors, lens, user, cate, text_features,
           event_types, cate_dict, params):
    p = params
    f32 = jnp.float32

    dv = jnp.transpose(divided, (0, 1, 3, 2))
    tm = (jnp.arange(T)[None, :] < lens[:, None]).astype(f32).reshape(B, 1, T)
    ug = (user[:, 0:1] == jnp.arange(2)[None, :]).astype(f32).reshape(B, 1, 2)
    ua = (user[:, 1:2] == jnp.arange(13)[None, :]).astype(f32).reshape(B, 1, 13)
    uc = (user[:, 2:3] == jnp.arange(10)[None, :]).astype(f32).reshape(B, 1, 10)
    cdoh = (cate_dict[:, None] == jnp.arange(CATE_NUM)[None, :]).astype(f32)

    row = lambda x: x.reshape(1, -1)
    col = lambda x: x.reshape(-1, 1)

    per_batch = [
        (code_x, (1, T, N)),
        (neighbors, (1, T, N)),
        (dv, (1, T, 3, N)),
        (cate, (1, T, CATE_NUM)),
        (text_features, (1, T, TEXT_DIM)),
        (tm, (1, 1, T)),
        (ug, (1, 1, 2)),
        (ua, (1, 1, 13)),
        (uc, (1, 1, 10)),
    ]
    bcast = [
        cdoh, p['c_emb'], p['n_emb'], p['u_emb'], p['cate_emb'],
        p['adj'], p['cate_adj'], p['W_g'], row(p['b_g']),
        p['W_ir'], p['W_iz'], p['W_in'],
        p['W_hr'], p['W_hz'], p['W_hn'],
        row(p['b_r']), row(p['b_z']), row(p['b_n']),
        p['W_q'], p['W_k'], p['W_v'], p['W_sa'],
        p['W_da'], row(p['b_da']), col(p['ctx_da']),
        p['W_da2'], row(p['b_da2']), col(p['ctx_da2']),
        p['us_gender'], p['us_age'], p['us_cluster'],
    ]

    in_specs = [pl.BlockSpec(bs, lambda i, _nd=len(bs): (i,) + (0,) * (_nd - 1))
                for _, bs in per_batch]
    in_specs += [_bcast_spec(a.shape) for a in bcast]

    O = pl.pallas_call(
        _fwd_kernel,
        grid=(B,),
        in_specs=in_specs,
        out_specs=pl.BlockSpec((1, 1, CAT_SIZE), lambda i: (i, 0, 0)),
        out_shape=jax.ShapeDtypeStruct((B, 1, CAT_SIZE), f32),
        compiler_params=pltpu.CompilerParams(
            dimension_semantics=("arbitrary",)),
    )(*[a for a, _ in per_batch], *bcast)

    out = pl.pallas_call(
        _bn_kernel,
        out_shape=jax.ShapeDtypeStruct((B, OUT_SIZE), f32),
    )(O.reshape(B, CAT_SIZE), row(p['bn_gamma']), row(p['bn_beta']),
      p['W_c'], row(p['b_c']))
    return out
